# baseline (speedup 1.0000x reference)
import jax
import jax.numpy as jnp
from jax import lax
from jax.experimental import pallas as pl
from jax.experimental.pallas import tpu as pltpu
from jax.experimental.pallas import tpu_sc as plsc

_NC = 2
_NS = 16
_NW = _NC * _NS

_B = 4096 * 200
_D = 64
_BPW = _B // _NW
_CH = 1024
_NCHUNK = _BPW // _CH


def _sc_gather(idx_hbm, table_hbm, out_hbm, idx_v, rows_v, sem):
    wid = lax.axis_index("s") * _NC + lax.axis_index("c")
    base = wid * _BPW

    def body(g, carry):
        off = base + g * _CH
        pltpu.sync_copy(idx_hbm.at[pl.ds(off, _CH)], idx_v)
        pltpu.async_copy(table_hbm.at[idx_v], rows_v, sem).wait()
        pltpu.sync_copy(rows_v, out_hbm.at[pl.ds(off, _CH)])
        return carry

    lax.fori_loop(0, _NCHUNK, body, 0)


@jax.jit
def kernel(token_ids, table):
    idx = token_ids.reshape(-1).astype(jnp.int32)
    mesh = plsc.VectorSubcoreMesh(
        core_axis_name="c", subcore_axis_name="s",
        num_cores=_NC, num_subcores=_NS,
    )
    out = pl.kernel(
        _sc_gather,
        out_type=jax.ShapeDtypeStruct((_B, _D), jnp.float32),
        mesh=mesh,
        scratch_types=[
            pltpu.VMEM((_CH,), jnp.int32),
            pltpu.VMEM((_CH, _D), jnp.float32),
            pltpu.SemaphoreType.DMA,
        ],
        compiler_params=pltpu.CompilerParams(use_tc_tiling_on_sc=False),
    )(idx, table)
    return out.reshape(token_ids.shape + (_D,))


# speedup vs baseline: 1.0164x; 1.0164x over previous
import jax
import jax.numpy as jnp
from jax import lax
from jax.experimental import pallas as pl
from jax.experimental.pallas import tpu as pltpu
from jax.experimental.pallas import tpu_sc as plsc

_NC = 2
_NS = 16
_NW = _NC * _NS

_B = 4096 * 200
_D = 64
_BPW = _B // _NW
_NBUF = 4
_CH = 400
_NCHUNK = _BPW // _CH
_NGROUP = _NCHUNK // _NBUF


def _sc_gather(idx_hbm, table_hbm, out_hbm, idx_v, rows_v, *sems):
    gsem = sems[:_NBUF]
    wsem = sems[_NBUF:]
    wid = lax.axis_index("s") * _NC + lax.axis_index("c")
    base = wid * _BPW

    def group(q, carry):
        descs = []
        for b in range(_NBUF):
            g = q * _NBUF + b
            off = base + g * _CH

            @pl.when(q > 0)
            def _():
                prev_off = off - _NBUF * _CH
                pltpu.make_async_copy(
                    rows_v.at[b], out_hbm.at[pl.ds(prev_off, _CH)], wsem[b]
                ).wait()

            pltpu.sync_copy(idx_hbm.at[pl.ds(off, _CH)], idx_v.at[b])
            descs.append(
                pltpu.async_copy(table_hbm.at[idx_v.at[b]], rows_v.at[b], gsem[b])
            )
        for b in range(_NBUF):
            g = q * _NBUF + b
            off = base + g * _CH
            descs[b].wait()
            pltpu.async_copy(rows_v.at[b], out_hbm.at[pl.ds(off, _CH)], wsem[b])
        return carry

    lax.fori_loop(0, _NGROUP, group, 0)

    for b in range(_NBUF):
        g = (_NGROUP - 1) * _NBUF + b
        off = base + g * _CH
        pltpu.make_async_copy(
            rows_v.at[b], out_hbm.at[pl.ds(off, _CH)], wsem[b]
        ).wait()


@jax.jit
def kernel(token_ids, table):
    idx = token_ids.reshape(-1).astype(jnp.int32)
    mesh = plsc.VectorSubcoreMesh(
        core_axis_name="c", subcore_axis_name="s",
        num_cores=_NC, num_subcores=_NS,
    )
    out = pl.kernel(
        _sc_gather,
        out_type=jax.ShapeDtypeStruct((_B, _D), jnp.float32),
        mesh=mesh,
        scratch_types=(
            [pltpu.VMEM((_NBUF, _CH), jnp.int32),
             pltpu.VMEM((_NBUF, _CH, _D), jnp.float32)]
            + [pltpu.SemaphoreType.DMA] * (2 * _NBUF)
        ),
        compiler_params=pltpu.CompilerParams(use_tc_tiling_on_sc=False),
    )(idx, table)
    return out.reshape(token_ids.shape + (_D,))
